# baseline (speedup 1.0000x reference)
import functools
import jax
import jax.numpy as jnp
import numpy as np
from jax.experimental import pallas as pl

N = 10000
E = 160000
D_IN = 128
D_H = 256
K = 512
CC = 0.25
MASK_RATIO = 0.15
SCE = 2


def _normalize(x, eps=1e-12):
    n = jnp.linalg.norm(x, axis=-1, keepdims=True)
    return x / jnp.maximum(n, eps)


def _vq_body(xn_ref, cn_ref, idx_ref):
    xn = xn_ref[...]
    cn = cn_ref[...]
    dot = jax.lax.dot_general(xn, cn, (((1,), (1,)), ((), ())),
                              preferred_element_type=jnp.float32)
    a = jnp.sum(xn * xn, axis=1, keepdims=True)
    s = jnp.sum(cn * cn, axis=1)[None, :]
    d = a + s - 2.0 * dot
    idx_ref[0, 0, :] = jnp.argmin(d, axis=1).astype(jnp.int32)


def _vq_argmin(xn, cn):
    BR = 1000
    grid = (N // BR,)
    idx = pl.pallas_call(
        _vq_body,
        grid=grid,
        in_specs=[
            pl.BlockSpec((BR, D_H), lambda i: (i, 0)),
            pl.BlockSpec((K, D_H), lambda i: (0, 0)),
        ],
        out_specs=pl.BlockSpec((1, 1, BR), lambda i: (i, 0, 0)),
        out_shape=jax.ShapeDtypeStruct((N // BR, 1, BR), jnp.int32),
    )(xn, cn)
    return idx.reshape(N)


def _graph_conv(x, src, dst, W, b):
    deg_out = jnp.zeros((N,), jnp.float32).at[src].add(1.0)
    deg_in = jnp.zeros((N,), jnp.float32).at[dst].add(1.0)
    ns = jnp.where(deg_out > 0, 1.0 / jnp.sqrt(jnp.maximum(deg_out, 1e-9)), 0.0)
    nd = jnp.where(deg_in > 0, 1.0 / jnp.sqrt(jnp.maximum(deg_in, 1e-9)), 0.0)
    h = x @ W
    m = h[src] * ns[src][:, None]
    agg = jnp.zeros((N, h.shape[1]), h.dtype).at[dst].add(m)
    return agg * nd[:, None] + b


def _hetero(x, edges, convs):
    out = 0.0
    for r in ('SEQ', 'KNN', 'DIS'):
        src, dst = edges[r][0], edges[r][1]
        out = out + _graph_conv(x, src, dst, convs[r]['W'], convs[r]['b'])
    return out


def _bn(x, g, b):
    mu = jnp.mean(x, axis=0)
    var = jnp.var(x, axis=0)
    return (x - mu) / jnp.sqrt(var + 1e-5) * g + b


def _encode(x, edges, enc):
    for l in range(2):
        x = _hetero(x, edges, enc['convs'][l])
        x = x @ enc['fcs'][l]['W'] + enc['fcs'][l]['b']
        x = _bn(jax.nn.relu(x), enc['bns'][l]['g'], enc['bns'][l]['b'])
    return x


def _decode(e, edges, dec):
    x = _hetero(e, edges, dec['convs'][0])
    x = x @ dec['fcs'][0]['W'] + dec['fcs'][0]['b']
    x = _bn(jax.nn.relu(x), dec['bns'][0]['g'], dec['bns'][0]['b'])
    x = _hetero(x, edges, dec['convs'][1])
    x = x @ dec['fcs'][1]['W'] + dec['fcs'][1]['b']
    return x


def kernel(x, edge_index_seq, edge_index_knn, edge_index_dis, mask, params, codebook):
    edges = {'SEQ': edge_index_seq, 'KNN': edge_index_knn, 'DIS': edge_index_dis}
    maskf = mask.astype(jnp.float32)
    x_in = x
    z = _encode(x_in, edges, params['enc'])
    xn = _normalize(z)
    cn = _normalize(codebook)
    idx = _vq_argmin(xn, cn)
    quant = _normalize(jnp.take(codebook, idx, axis=0))
    q_loss = jnp.mean((quant - xn) ** 2)
    e_loss = jnp.mean((xn - quant) ** 2)
    e_q_loss = q_loss + CC * e_loss
    e = xn + (quant - xn)
    x_recon = _decode(e, edges, params['dec'])
    recon_loss = jnp.mean((x_recon - x_in) ** 2)
    mi = maskf[idx]
    e_masked = e * (1.0 - mi)[:, None]
    x_mask_recon = _decode(e_masked, edges, params['dec'])
    a = _normalize(x_mask_recon)
    b = _normalize(x_in)
    per_node = (1.0 - jnp.sum(a * b, axis=-1)) ** SCE
    mask_loss = jnp.sum(per_node * mi) / (jnp.sum(mi) + 1e-12)
    return z, e_masked, e_q_loss, recon_loss, mask_loss


# speedup vs baseline: 1.6716x; 1.6716x over previous
import functools
import jax
import jax.numpy as jnp
from jax import lax
from jax.experimental import pallas as pl
from jax.experimental.pallas import tpu as pltpu
from jax.experimental.pallas import tpu_sc as plsc

N = 10000
E = 160000
D_IN = 128
D_H = 256
K = 512
CC = 0.25
SCE = 2

NTILE = 16
CHUNK = 125
ROWS_PER_TILE = 80
NROW2D = E // CHUNK
NPAD = 10240
NPT = NPAD // NTILE

_mesh = plsc.VectorSubcoreMesh(core_axis_name="c", subcore_axis_name="s")


HROW = 5120
AROW = 5248
JUNK = 5192
CCHUNK = 128
CROWS = 80
EPAD = 16 * CROWS * CCHUNK


def _remap(idx_d, g, p, idx_r):
    for j in range(CCHUNK // 16):
        v = idx_d[g, pl.ds(16 * j, 16)]
        lo = v - jnp.int32(p * HROW)
        ok = (lo >= 0) & (lo < HROW)
        idx_r[pl.ds(16 * j, 16)] = jnp.where(ok, lo, jnp.int32(JUNK))


def _conv_body(h0, h1, src2d, dst2d, zeros2d, out0, out1,
               idx_s, idx_d, idx_r0, idx_r1, rows0, rows1, acc, sem0, sem1):
    s = lax.axis_index("s")
    c = lax.axis_index("c")

    pltpu.sync_copy(src2d.at[pl.ds(s * CROWS, CROWS)], idx_s)
    pltpu.sync_copy(dst2d.at[pl.ds(s * CROWS, CROWS)], idx_d)

    def row_pass(p, h_hbm, out_hbm):
        pltpu.sync_copy(zeros2d, acc.at[pl.ds(s * (AROW // 16), AROW // 16)])
        plsc.subcore_barrier()

        pltpu.async_copy(h_hbm.at[idx_s.at[0]], rows0, sem0)
        pltpu.async_copy(h_hbm.at[idx_s.at[1]], rows1, sem1)

        def step(i, _):
            g = 2 * i
            pltpu.make_async_copy(h_hbm.at[idx_s.at[g]], rows0, sem0).wait()
            _remap(idx_d, g, p, idx_r0)
            pltpu.sync_copy(rows0, acc.at[idx_r0], add=True)

            @pl.when(g + 2 < CROWS)
            def _():
                pltpu.async_copy(h_hbm.at[idx_s.at[g + 2]], rows0, sem0)

            pltpu.make_async_copy(h_hbm.at[idx_s.at[g + 1]], rows1, sem1).wait()
            _remap(idx_d, g + 1, p, idx_r1)
            pltpu.sync_copy(rows1, acc.at[idx_r1], add=True)

            @pl.when(g + 3 < CROWS)
            def _():
                pltpu.async_copy(h_hbm.at[idx_s.at[g + 3]], rows1, sem1)
            return 0

        lax.fori_loop(0, CROWS // 2, step, 0)
        plsc.subcore_barrier()
        pltpu.sync_copy(acc.at[pl.ds(s * 320, 320)],
                        out_hbm.at[pl.ds(p * HROW + s * 320, 320)])
        plsc.subcore_barrier()

    def col_half(h_hbm, out_hbm):
        row_pass(0, h_hbm, out_hbm)
        row_pass(1, h_hbm, out_hbm)

    @pl.when(c == 0)
    def _():
        col_half(h0, out0)

    @pl.when(c == 1)
    def _():
        col_half(h1, out1)


def _sc_conv(h0, h1, src2d, dst2d, zeros2d):
    f = pl.kernel(
        _conv_body,
        out_type=[jax.ShapeDtypeStruct((NPAD, 128), jnp.float32)] * 2,
        mesh=_mesh,
        scratch_types=[
            pltpu.VMEM((CROWS, CCHUNK), jnp.int32),
            pltpu.VMEM((CROWS, CCHUNK), jnp.int32),
            pltpu.VMEM((CCHUNK,), jnp.int32),
            pltpu.VMEM((CCHUNK,), jnp.int32),
            pltpu.VMEM((CCHUNK, 128), jnp.float32),
            pltpu.VMEM((CCHUNK, 128), jnp.float32),
            pltpu.VMEM_SHARED((AROW, 128), jnp.float32),
            pltpu.SemaphoreType.DMA,
            pltpu.SemaphoreType.DMA,
        ],
    )
    return f(h0, h1, src2d, dst2d, zeros2d)


def _deg_body(i0, i1, i2, i3, i4, i5, zeros640, o0, o1, o2, o3, o4, o5,
              idxblk, ones, a0, a1, a2, sem):
    s = lax.axis_index("s")
    c = lax.axis_index("c")

    for j in range(8):
        ones[pl.ds(16 * j, 16)] = jnp.ones((16,), jnp.float32)

    def histo(idx_hbm, acc):
        pltpu.sync_copy(zeros640, acc.at[pl.ds(s * 640, 640)])
        pltpu.sync_copy(idx_hbm.at[pl.ds(s * ROWS_PER_TILE, ROWS_PER_TILE)],
                        idxblk)
        plsc.subcore_barrier()

        def step(i, _):
            for j in range(10):
                pltpu.async_copy(ones.at[pl.ds(0, CHUNK)],
                                 acc.at[idxblk.at[10 * i + j]], sem, add=True)
            for j in range(10):
                pltpu.make_async_copy(ones.at[pl.ds(0, CHUNK)],
                                      acc.at[idxblk.at[10 * i + j]], sem).wait()
            return 0

        lax.fori_loop(0, ROWS_PER_TILE // 10, step, 0)
        plsc.subcore_barrier()

    def emit(idx_hbm, acc, out_hbm):
        histo(idx_hbm, acc)
        pltpu.sync_copy(acc.at[pl.ds(s * 640, 640)],
                        out_hbm.at[pl.ds(s * 640, 640)])

    @pl.when(c == 0)
    def _():
        emit(i0, a0, o0)
        emit(i1, a1, o1)
        emit(i2, a2, o2)

    @pl.when(c == 1)
    def _():
        emit(i3, a0, o3)
        emit(i4, a1, o4)
        emit(i5, a2, o5)


def _sc_degrees(idx6, zeros640):
    f = pl.kernel(
        _deg_body,
        out_type=[jax.ShapeDtypeStruct((NPAD,), jnp.float32)] * 6,
        mesh=_mesh,
        scratch_types=[
            pltpu.VMEM((ROWS_PER_TILE, CHUNK), jnp.int32),
            pltpu.VMEM((128,), jnp.float32),
            pltpu.VMEM_SHARED((NPAD,), jnp.float32),
            pltpu.VMEM_SHARED((NPAD,), jnp.float32),
            pltpu.VMEM_SHARED((NPAD,), jnp.float32),
            pltpu.SemaphoreType.DMA,
        ],
    )
    return f(*idx6, zeros640)


def _vq_body(xn_ref, cn_ref, idx_ref):
    xn = xn_ref[...]
    cn = cn_ref[...]
    dot = lax.dot_general(xn, cn, (((1,), (1,)), ((), ())),
                          preferred_element_type=jnp.float32)
    a = jnp.sum(xn * xn, axis=1, keepdims=True)
    sc = jnp.sum(cn * cn, axis=1)[None, :]
    d = a + sc - 2.0 * dot
    idx_ref[0, 0, :] = jnp.argmin(d, axis=1).astype(jnp.int32)


def _vq_argmin(xn, cn):
    BR = 1000
    idx = pl.pallas_call(
        _vq_body,
        grid=(N // BR,),
        in_specs=[
            pl.BlockSpec((BR, D_H), lambda i: (i, 0)),
            pl.BlockSpec((K, D_H), lambda i: (0, 0)),
        ],
        out_specs=pl.BlockSpec((1, 1, BR), lambda i: (i, 0, 0)),
        out_shape=jax.ShapeDtypeStruct((N // BR, 1, BR), jnp.int32),
    )(xn, cn)
    return idx.reshape(N)


def _normalize(x, eps=1e-12):
    n = jnp.linalg.norm(x, axis=-1, keepdims=True)
    return x / jnp.maximum(n, eps)


def _norm_coeff(deg):
    return jnp.where(deg > 0, 1.0 / jnp.sqrt(jnp.maximum(deg, 1e-9)), 0.0)


def _bn(x, g, b):
    mu = jnp.mean(x, axis=0)
    var = jnp.var(x, axis=0)
    return (x - mu) / jnp.sqrt(var + 1e-5) * g + b


def _hetero_sc(x, conv_params, ed, zeros2d):
    out = 0.0
    for r in ('SEQ', 'KNN', 'DIS'):
        W, b = conv_params[r]['W'], conv_params[r]['b']
        src2d, dst2d, _, _, ns, nd = ed[r]
        h = (x @ W) * ns[:, None]
        a0, a1 = _sc_conv(h[:, :128], h[:, 128:], src2d, dst2d, zeros2d)
        agg = jnp.concatenate([a0[:N], a1[:N]], axis=1)
        out = out + (agg * nd[:, None] + b)
    return out


def _hetero_exact(x, conv_params, ed):
    out = 0.0
    for r in ('SEQ', 'KNN', 'DIS'):
        W, b = conv_params[r]['W'], conv_params[r]['b']
        src, dst, ns, nd = ed[r][2], ed[r][3], ed[r][4], ed[r][5]
        h = x @ W
        m = h[src] * ns[src][:, None]
        agg = jnp.zeros((N, h.shape[1]), h.dtype).at[dst].add(m)
        out = out + (agg * nd[:, None] + b)
    return out


def _encode(x, enc, ed):
    for l in range(2):
        x = _hetero_exact(x, enc['convs'][l], ed)
        x = x @ enc['fcs'][l]['W'] + enc['fcs'][l]['b']
        x = _bn(jax.nn.relu(x), enc['bns'][l]['g'], enc['bns'][l]['b'])
    return x


def _decode(e, dec, ed, zeros2d):
    x = _hetero_sc(e, dec['convs'][0], ed, zeros2d)
    x = x @ dec['fcs'][0]['W'] + dec['fcs'][0]['b']
    x = _bn(jax.nn.relu(x), dec['bns'][0]['g'], dec['bns'][0]['b'])
    x = _hetero_sc(x, dec['convs'][1], ed, zeros2d)
    x = x @ dec['fcs'][1]['W'] + dec['fcs'][1]['b']
    return x


def kernel(x, edge_index_seq, edge_index_knn, edge_index_dis, mask, params,
           codebook):
    maskf = mask.astype(jnp.float32)
    zeros2d = jnp.zeros((AROW // 16, 128), jnp.float32)
    zeros640 = jnp.zeros((640,), jnp.float32)
    pad_src = jnp.arange(EPAD - E, dtype=jnp.int32) % N
    pad_dst = jnp.full((EPAD - E,), 1 << 30, jnp.int32)

    e2d, ec = {}, {}
    for r, ei in (('SEQ', edge_index_seq), ('KNN', edge_index_knn),
                  ('DIS', edge_index_dis)):
        e2d[r] = (ei[0].reshape(NROW2D, CHUNK), ei[1].reshape(NROW2D, CHUNK))
        ec[r] = (jnp.concatenate([ei[0], pad_src]).reshape(EPAD // CCHUNK, CCHUNK),
                 jnp.concatenate([ei[1], pad_dst]).reshape(EPAD // CCHUNK, CCHUNK))

    idx6 = [e2d['SEQ'][0], e2d['SEQ'][1], e2d['KNN'][0],
            e2d['KNN'][1], e2d['DIS'][0], e2d['DIS'][1]]
    degs = _sc_degrees(idx6, zeros640)
    srcdst = {'SEQ': edge_index_seq, 'KNN': edge_index_knn,
              'DIS': edge_index_dis}
    ed = {}
    for k, r in enumerate(('SEQ', 'KNN', 'DIS')):
        ns = _norm_coeff(degs[2 * k][:N])
        nd = _norm_coeff(degs[2 * k + 1][:N])
        ed[r] = (ec[r][0], ec[r][1], srcdst[r][0], srcdst[r][1], ns, nd)

    x_in = x
    z = _encode(x_in, params['enc'], ed)
    xn = _normalize(z)
    cn = _normalize(codebook)
    idx = _vq_argmin(xn, cn)
    onehot = (idx[:, None] == jnp.arange(K)[None, :]).astype(jnp.float32)
    quant = onehot @ cn
    q_loss = jnp.mean((quant - xn) ** 2)
    e_q_loss = q_loss + CC * q_loss
    e = xn + (quant - xn)
    x_recon = _decode(e, params['dec'], ed, zeros2d)
    recon_loss = jnp.mean((x_recon - x_in) ** 2)
    mi = onehot @ maskf
    e_masked = e * (1.0 - mi)[:, None]
    x_mask_recon = _decode(e_masked, params['dec'], ed, zeros2d)
    a = _normalize(x_mask_recon)
    b = _normalize(x_in)
    per_node = (1.0 - jnp.sum(a * b, axis=-1)) ** SCE
    mask_loss = jnp.sum(per_node * mi) / (jnp.sum(mi) + 1e-12)
    return z, e_masked, e_q_loss, recon_loss, mask_loss
